# baseline (speedup 1.0000x reference)
import functools

import jax
import jax.numpy as jnp
from jax import lax
from jax.experimental import pallas as pl
from jax.experimental.pallas import tpu as pltpu
from jax.experimental.pallas import tpu_sc as plsc

B, Wn, F, H, D, K, ALPHA = 4, 4096, 32, 2, 32, 16, 0.2
N = B * Wn
HD = H * D
NB = 2 * K + 1
NW = 32
NPW = N // NW
PADW = 128
WIN = NPW + 2 * PADW
OFB = PADW - K
L16 = 16
NG = NPW // L16
NEG = -1e30

PBLK = 2048


def _proj_kernel(x_ref, ws_ref, wd_ref, fsT_ref, fdT_ref):
    xb = x_ref[...]
    dn = (((0,), (1,)), ((), ()))
    fsT_ref[...] = lax.dot_general(ws_ref[...], xb, dn,
                                   preferred_element_type=jnp.float32)
    fdT_ref[...] = lax.dot_general(wd_ref[...], xb, dn,
                                   preferred_element_type=jnp.float32)


_mesh = plsc.VectorSubcoreMesh(core_axis_name="c", subcore_axis_name="s")


@functools.partial(
    pl.kernel,
    out_type=jax.ShapeDtypeStruct((D, N), jnp.float32),
    mesh=_mesh,
    compiler_params=pltpu.CompilerParams(use_tc_tiling_on_sc=False,
                                         needs_layout_passes=False),
    scratch_types=[
        pltpu.VMEM((HD, WIN), jnp.float32),
        pltpu.VMEM((HD, NPW), jnp.float32),
        pltpu.VMEM((HD, 128), jnp.float32),
        pltpu.VMEM((H, NB, L16), jnp.float32),
        pltpu.VMEM((H, NB, L16), jnp.float32),
        pltpu.VMEM((D, NPW), jnp.float32),
    ],
)
def _sc_attn(fsT_hbm, fdT_hbm, ab_hbm, out_hbm,
             fs_v, fd_v, a_v, lg_v, wg_v, ov_v):
    wid = lax.axis_index("s") * 2 + lax.axis_index("c")
    n0g = wid * NPW
    pltpu.sync_copy(fsT_hbm.at[:, pl.ds(n0g, WIN)], fs_v)
    pltpu.sync_copy(fdT_hbm.at[:, pl.ds(n0g, NPW)], fd_v)
    pltpu.sync_copy(ab_hbm, a_v)

    lanes = jnp.arange(L16, dtype=jnp.int32)
    zero = jnp.zeros((L16,), jnp.float32)

    def group_body(g, carry):
        n0 = pl.multiple_of(g * L16, L16)
        pvec = (n0g + n0) % Wn + lanes

        for h in range(H):
            for o in range(NB):
                lg_v[h, o, :] = zero

        for h in range(H):
            def d_body(d, c, h=h):
                hd = h * D + d
                fdv = fd_v[hd, pl.ds(n0, L16)]
                av = a_v[hd, pl.ds(0, L16)]
                row = jnp.full((L16,), hd, jnp.int32)
                col0 = (n0 + OFB) + lanes
                for o in range(NB):
                    fsv = plsc.load_gather(fs_v, [row, col0 + o])
                    t = fsv + fdv
                    t = jnp.maximum(t, ALPHA * t)
                    plsc.addupdate(lg_v.at[h, o, :], t * av)
                return c
            lax.fori_loop(0, D, d_body, 0)

        for h in range(H):
            mx = jnp.full((L16,), NEG, jnp.float32)
            for o in range(NB):
                po = pvec + (o - K)
                m = (po >= 0) & (po <= Wn - 1)
                lm = jnp.where(m, lg_v[h, o, :], NEG)
                lg_v[h, o, :] = lm
                mx = jnp.maximum(mx, lm)
            den = zero
            for o in range(NB):
                ex = jnp.exp(lg_v[h, o, :] - mx)
                wg_v[h, o, :] = ex
                den = den + ex
            inv = 0.5 / den
            for o in range(NB):
                wg_v[h, o, :] = wg_v[h, o, :] * inv

        for d in range(D):
            ov_v[d, pl.ds(n0, L16)] = zero

        def o_body(o, c):
            w0 = wg_v[0, o, :]
            w1 = wg_v[1, o, :]
            col = (n0 + OFB) + o + lanes
            for d in range(D):
                f0 = plsc.load_gather(fs_v, [jnp.full((L16,), d, jnp.int32), col])
                f1 = plsc.load_gather(fs_v, [jnp.full((L16,), D + d, jnp.int32), col])
                plsc.addupdate(ov_v.at[d, pl.ds(n0, L16)], f0 * w0 + f1 * w1)
            return c
        lax.fori_loop(0, NB, o_body, 0)
        return carry

    lax.fori_loop(0, NG, group_body, 0)
    pltpu.sync_copy(ov_v, out_hbm.at[:, pl.ds(n0g, NPW)])


@jax.jit
def _run(x, W_src, W_dst, attn_a, bias):
    nf = x.reshape(N, F)
    fsT, fdT = pl.pallas_call(
        _proj_kernel,
        grid=(N // PBLK,),
        in_specs=[
            pl.BlockSpec((PBLK, F), lambda i: (i, 0)),
            pl.BlockSpec((F, HD), lambda i: (0, 0)),
            pl.BlockSpec((F, HD), lambda i: (0, 0)),
        ],
        out_specs=[
            pl.BlockSpec((HD, PBLK), lambda i: (0, i)),
            pl.BlockSpec((HD, PBLK), lambda i: (0, i)),
        ],
        out_shape=[
            jax.ShapeDtypeStruct((HD, N), jnp.float32),
            jax.ShapeDtypeStruct((HD, N), jnp.float32),
        ],
    )(nf, W_src, W_dst)
    fsT_pad = jnp.pad(fsT, ((0, 0), (PADW, PADW)))
    a_b = jnp.broadcast_to(attn_a.reshape(HD, 1), (HD, 128))
    outT = _sc_attn(fsT_pad, fdT, a_b)
    out = outT.T + bias.reshape(H, D).mean(axis=0)[None, :]
    return out.reshape(B, Wn, D)


def kernel(x, W_src, W_dst, attn_a, bias, src, dst):
    del src, dst
    return _run(x, W_src, W_dst, attn_a, bias)


# speedup vs baseline: 3.7694x; 3.7694x over previous
import functools

import jax
import jax.numpy as jnp
from jax import lax
from jax.experimental import pallas as pl
from jax.experimental.pallas import tpu as pltpu
from jax.experimental.pallas import tpu_sc as plsc

B, Wn, F, H, D, K, ALPHA = 4, 4096, 32, 2, 32, 16, 0.2
N = B * Wn
HD = H * D
NB = 2 * K + 1
NW = 32
NPW = N // NW
PADW = 128
WIN = NPW + 2 * PADW
OFB = PADW - K
L16 = 16
NG = NPW // L16
NEG = -1e30

PBLK = 2048


def _proj_kernel(x_ref, ws_ref, wd_ref, fsT_ref, fdT_ref):
    xb = x_ref[...]
    dn = (((0,), (1,)), ((), ()))
    fsT_ref[...] = lax.dot_general(ws_ref[...], xb, dn,
                                   preferred_element_type=jnp.float32)
    fdT_ref[...] = lax.dot_general(wd_ref[...], xb, dn,
                                   preferred_element_type=jnp.float32)


_mesh = plsc.VectorSubcoreMesh(core_axis_name="c", subcore_axis_name="s")


@functools.partial(
    pl.kernel,
    out_type=jax.ShapeDtypeStruct((D, N), jnp.float32),
    mesh=_mesh,
    compiler_params=pltpu.CompilerParams(use_tc_tiling_on_sc=False,
                                         needs_layout_passes=False),
    scratch_types=[
        pltpu.VMEM((HD, WIN), jnp.float32),
        pltpu.VMEM((HD, NPW), jnp.float32),
        pltpu.VMEM((HD, 128), jnp.float32),
        pltpu.VMEM((H, NB, L16), jnp.float32),
        pltpu.VMEM((H, NB, L16), jnp.float32),
        pltpu.VMEM((D, NPW), jnp.float32),
    ],
)
def _sc_attn(fsT_hbm, fdT_hbm, ab_hbm, out_hbm,
             fs_v, fd_v, a_v, lg_v, wg_v, ov_v):
    wid = lax.axis_index("s") * 2 + lax.axis_index("c")
    n0g = wid * NPW
    pltpu.sync_copy(fsT_hbm.at[:, pl.ds(n0g, WIN)], fs_v)
    pltpu.sync_copy(fdT_hbm.at[:, pl.ds(n0g, NPW)], fd_v)
    pltpu.sync_copy(ab_hbm, a_v)

    lanes = jnp.arange(L16, dtype=jnp.int32)
    zero = jnp.zeros((L16,), jnp.float32)

    OCH = [(0, 8), (8, 8), (16, 8), (24, 8), (32, 1)]
    DCH = [(0, 16), (16, 16)]

    def group_body(g, carry):
        n0 = pl.multiple_of(g * L16, L16)
        pvec = (n0g + n0) % Wn + lanes
        col0 = (n0 + OFB) + lanes

        for h in range(H):
            mx = jnp.full((L16,), NEG, jnp.float32)
            for (co, cn) in OCH:
                def d_body(d, accs, h=h, co=co, cn=cn):
                    hd = h * D + d
                    fdv = fd_v[hd, pl.ds(n0, L16)]
                    av = a_v[hd, pl.ds(0, L16)]
                    row = jnp.full((L16,), hd, jnp.int32)
                    out = []
                    for i in range(cn):
                        fsv = plsc.load_gather(fs_v, [row, col0 + (co + i)])
                        t = fsv + fdv
                        t = jnp.maximum(t, ALPHA * t)
                        out.append(accs[i] + t * av)
                    return tuple(out)
                accs = lax.fori_loop(0, D, d_body, (zero,) * cn)
                for i in range(cn):
                    po = pvec + (co + i - K)
                    m = (po >= 0) & (po <= Wn - 1)
                    lm = jnp.where(m, accs[i], NEG)
                    lg_v[h, co + i, :] = lm
                    mx = jnp.maximum(mx, lm)

            den = zero
            for (co, cn) in OCH:
                exs = []
                for i in range(cn):
                    exs.append(jnp.exp(lg_v[h, co + i, :] - mx))
                for i in range(cn):
                    den = den + exs[i]
                    wg_v[h, co + i, :] = exs[i]
            inv = 0.5 / den
            for (co, cn) in OCH:
                vals = [wg_v[h, co + i, :] * inv for i in range(cn)]
                for i in range(cn):
                    wg_v[h, co + i, :] = vals[i]

        for (do, dn) in DCH:
            def o_body(o, accs, do=do, dn=dn):
                w0 = wg_v[0, o, :]
                w1 = wg_v[1, o, :]
                col = col0 + o
                out = []
                for i in range(dn):
                    f0 = plsc.load_gather(
                        fs_v, [jnp.full((L16,), do + i, jnp.int32), col])
                    f1 = plsc.load_gather(
                        fs_v, [jnp.full((L16,), D + do + i, jnp.int32), col])
                    out.append(accs[i] + (f0 * w0 + f1 * w1))
                return tuple(out)
            accs = lax.fori_loop(0, NB, o_body, (zero,) * dn)
            for i in range(dn):
                ov_v[do + i, pl.ds(n0, L16)] = accs[i]
        return carry

    lax.fori_loop(0, NG, group_body, 0)
    pltpu.sync_copy(ov_v, out_hbm.at[:, pl.ds(n0g, NPW)])


@jax.jit
def _run(x, W_src, W_dst, attn_a, bias):
    nf = x.reshape(N, F)
    fsT, fdT = pl.pallas_call(
        _proj_kernel,
        grid=(N // PBLK,),
        in_specs=[
            pl.BlockSpec((PBLK, F), lambda i: (i, 0)),
            pl.BlockSpec((F, HD), lambda i: (0, 0)),
            pl.BlockSpec((F, HD), lambda i: (0, 0)),
        ],
        out_specs=[
            pl.BlockSpec((HD, PBLK), lambda i: (0, i)),
            pl.BlockSpec((HD, PBLK), lambda i: (0, i)),
        ],
        out_shape=[
            jax.ShapeDtypeStruct((HD, N), jnp.float32),
            jax.ShapeDtypeStruct((HD, N), jnp.float32),
        ],
    )(nf, W_src, W_dst)
    fsT_pad = jnp.pad(fsT, ((0, 0), (PADW, PADW)))
    a_b = jnp.broadcast_to(attn_a.reshape(HD, 1), (HD, 128))
    outT = _sc_attn(fsT_pad, fdT, a_b)
    out = outT.T + bias.reshape(H, D).mean(axis=0)[None, :]
    return out.reshape(B, Wn, D)


def kernel(x, W_src, W_dst, attn_a, bias, src, dst):
    del src, dst
    return _run(x, W_src, W_dst, attn_a, bias)


# speedup vs baseline: 3.9871x; 1.0578x over previous
import functools

import jax
import jax.numpy as jnp
from jax import lax
from jax.experimental import pallas as pl
from jax.experimental.pallas import tpu as pltpu
from jax.experimental.pallas import tpu_sc as plsc

B, Wn, F, H, D, K, ALPHA = 4, 4096, 32, 2, 32, 16, 0.2
N = B * Wn
HD = H * D
NB = 2 * K + 1
NEG = -1e30
PADW = 128
NP = N + 2 * PADW

NT = 2048
NW = 32
NPW = (N - NT) // NW
L16 = 16
NG = NPW // L16
WIN = NPW + 2 * K

PBLK = 1664
TBLK = 512


def _proj_kernel(x_ref, ws_ref, wd_ref, fsT_ref, fdT_ref):
    xb = x_ref[...]
    dn = (((0,), (1,)), ((), ()))
    fsT_ref[...] = lax.dot_general(ws_ref[...], xb, dn,
                                   preferred_element_type=jnp.float32)
    fdT_ref[...] = lax.dot_general(wd_ref[...], xb, dn,
                                   preferred_element_type=jnp.float32)


def _band_kernel(xp_ref, ws_ref, wd_ref, a_ref, out_ref,
                 fs_ref, w0_ref, w1_ref):
    pid = pl.program_id(0)
    x_halo = xp_ref[pl.ds(pid * TBLK + PADW - K, TBLK + 2 * K), :]
    fs_ref[...] = jnp.dot(x_halo, ws_ref[...],
                          preferred_element_type=jnp.float32)
    fd = jnp.dot(x_halo[K:K + TBLK], wd_ref[...],
                 preferred_element_type=jnp.float32)
    a = a_ref[...]

    p0 = (pid % (Wn // TBLK)) * TBLK
    p = p0 + jax.lax.broadcasted_iota(jnp.int32, (TBLK, 1), 0)

    for o in range(NB):
        off = o - K
        e = fs_ref[o:o + TBLK] + fd
        e = jnp.where(e > 0, e, ALPHA * e)
        ea = e * a
        l0 = jnp.sum(ea[:, :D], axis=1, keepdims=True)
        l1 = jnp.sum(ea[:, D:], axis=1, keepdims=True)
        valid = (p + off >= 0) & (p + off <= Wn - 1)
        w0_ref[:, o:o + 1] = jnp.where(valid, l0, NEG)
        w1_ref[:, o:o + 1] = jnp.where(valid, l1, NEG)

    L0, L1 = w0_ref[...], w1_ref[...]
    m0 = jnp.max(L0, axis=1, keepdims=True)
    m1 = jnp.max(L1, axis=1, keepdims=True)
    ex0 = jnp.exp(L0 - m0)
    ex1 = jnp.exp(L1 - m1)
    w0_ref[...] = ex0 * (0.5 / jnp.sum(ex0, axis=1, keepdims=True))
    w1_ref[...] = ex1 * (0.5 / jnp.sum(ex1, axis=1, keepdims=True))

    acc = jnp.zeros((TBLK, D), dtype=jnp.float32)
    for o in range(NB):
        fs_sh = fs_ref[o:o + TBLK]
        acc = (acc + fs_sh[:, :D] * w0_ref[:, o:o + 1]
               + fs_sh[:, D:] * w1_ref[:, o:o + 1])
    out_ref[...] = acc


_mesh = plsc.VectorSubcoreMesh(core_axis_name="c", subcore_axis_name="s")


@functools.partial(
    pl.kernel,
    out_type=jax.ShapeDtypeStruct((D, N - NT), jnp.float32),
    mesh=_mesh,
    compiler_params=pltpu.CompilerParams(use_tc_tiling_on_sc=False,
                                         needs_layout_passes=False),
    scratch_types=[
        pltpu.VMEM((HD, WIN), jnp.float32),
        pltpu.VMEM((HD, NPW), jnp.float32),
        pltpu.VMEM((HD, 128), jnp.float32),
        pltpu.VMEM((H, NB, L16), jnp.float32),
        pltpu.VMEM((H, NB, L16), jnp.float32),
        pltpu.VMEM((D, NPW), jnp.float32),
    ],
)
def _sc_attn(fsT_hbm, fdT_hbm, ab_hbm, out_hbm,
             fs_v, fd_v, a_v, lg_v, wg_v, ov_v):
    wid = lax.axis_index("s") * 2 + lax.axis_index("c")
    n0g = NT + wid * NPW
    pltpu.sync_copy(fsT_hbm.at[:, pl.ds(n0g + PADW - K, WIN)], fs_v)
    pltpu.sync_copy(fdT_hbm.at[:, pl.ds(n0g + PADW, NPW)], fd_v)
    pltpu.sync_copy(ab_hbm, a_v)

    lanes = jnp.arange(L16, dtype=jnp.int32)
    zero = jnp.zeros((L16,), jnp.float32)

    OCH = [(0, 8), (8, 8), (16, 8), (24, 8), (32, 1)]
    DCH = [(0, 16), (16, 16)]

    def group_body(g, carry):
        n0 = pl.multiple_of(g * L16, L16)
        pvec = (n0g + n0) % Wn + lanes
        col0 = n0 + lanes

        for h in range(H):
            mx = jnp.full((L16,), NEG, jnp.float32)
            for (co, cn) in OCH:
                def d_body(d, accs, h=h, co=co, cn=cn):
                    hd = h * D + d
                    fdv = fd_v[hd, pl.ds(n0, L16)]
                    av = a_v[hd, pl.ds(0, L16)]
                    row = jnp.full((L16,), hd, jnp.int32)
                    out = []
                    for i in range(cn):
                        fsv = plsc.load_gather(fs_v, [row, col0 + (co + i)])
                        t = fsv + fdv
                        t = jnp.maximum(t, ALPHA * t)
                        out.append(accs[i] + t * av)
                    return tuple(out)
                accs = lax.fori_loop(0, D, d_body, (zero,) * cn)
                for i in range(cn):
                    po = pvec + (co + i - K)
                    m = (po >= 0) & (po <= Wn - 1)
                    lm = jnp.where(m, accs[i], NEG)
                    lg_v[h, co + i, :] = lm
                    mx = jnp.maximum(mx, lm)

            den = zero
            for (co, cn) in OCH:
                exs = []
                for i in range(cn):
                    exs.append(jnp.exp(lg_v[h, co + i, :] - mx))
                for i in range(cn):
                    den = den + exs[i]
                    wg_v[h, co + i, :] = exs[i]
            inv = 0.5 / den
            for (co, cn) in OCH:
                vals = [wg_v[h, co + i, :] * inv for i in range(cn)]
                for i in range(cn):
                    wg_v[h, co + i, :] = vals[i]

        for (do, dn) in DCH:
            def o_body(o, accs, do=do, dn=dn):
                w0 = wg_v[0, o, :]
                w1 = wg_v[1, o, :]
                col = col0 + o
                out = []
                for i in range(dn):
                    f0 = plsc.load_gather(
                        fs_v, [jnp.full((L16,), do + i, jnp.int32), col])
                    f1 = plsc.load_gather(
                        fs_v, [jnp.full((L16,), D + do + i, jnp.int32), col])
                    out.append(accs[i] + (f0 * w0 + f1 * w1))
                return tuple(out)
            accs = lax.fori_loop(0, NB, o_body, (zero,) * dn)
            for i in range(dn):
                ov_v[do + i, pl.ds(n0, L16)] = accs[i]
        return carry

    lax.fori_loop(0, NG, group_body, 0)
    pltpu.sync_copy(ov_v, out_hbm.at[:, pl.ds(wid * NPW, NPW)])


@jax.jit
def _run(x, W_src, W_dst, attn_a, bias):
    nf = x.reshape(N, F)
    xp = jnp.pad(nf, ((PADW, PADW), (0, 0)))
    fsT, fdT = pl.pallas_call(
        _proj_kernel,
        grid=(NP // PBLK,),
        in_specs=[
            pl.BlockSpec((PBLK, F), lambda i: (i, 0)),
            pl.BlockSpec((F, HD), lambda i: (0, 0)),
            pl.BlockSpec((F, HD), lambda i: (0, 0)),
        ],
        out_specs=[
            pl.BlockSpec((HD, PBLK), lambda i: (0, i)),
            pl.BlockSpec((HD, PBLK), lambda i: (0, i)),
        ],
        out_shape=[
            jax.ShapeDtypeStruct((HD, NP), jnp.float32),
            jax.ShapeDtypeStruct((HD, NP), jnp.float32),
        ],
    )(xp, W_src, W_dst)
    a_b = jnp.broadcast_to(attn_a.reshape(HD, 1), (HD, 128))
    outT_sc = _sc_attn(fsT, fdT, a_b)

    a_row = attn_a.reshape(1, HD)
    out_tc = pl.pallas_call(
        _band_kernel,
        grid=(NT // TBLK,),
        in_specs=[
            pl.BlockSpec((NP, F), lambda i: (0, 0)),
            pl.BlockSpec((F, HD), lambda i: (0, 0)),
            pl.BlockSpec((F, HD), lambda i: (0, 0)),
            pl.BlockSpec((1, HD), lambda i: (0, 0)),
        ],
        out_specs=pl.BlockSpec((TBLK, D), lambda i: (i, 0)),
        out_shape=jax.ShapeDtypeStruct((NT, D), jnp.float32),
        scratch_shapes=[
            pltpu.VMEM((TBLK + 2 * K, HD), jnp.float32),
            pltpu.VMEM((TBLK, NB), jnp.float32),
            pltpu.VMEM((TBLK, NB), jnp.float32),
        ],
    )(xp, W_src, W_dst, a_row)

    out = jnp.concatenate([out_tc, outT_sc.T], axis=0)
    out = out + bias.reshape(H, D).mean(axis=0)[None, :]
    return out.reshape(B, Wn, D)


def kernel(x, W_src, W_dst, attn_a, bias, src, dst):
    del src, dst
    return _run(x, W_src, W_dst, attn_a, bias)
